# Initial kernel scaffold; baseline (speedup 1.0000x reference)
import jax
import jax.numpy as jnp
from jax.experimental import pallas as pl


def kernel(pos, mass, velocity, edge_attr, edge_index, params):
    raise NotImplementedError("write your pallas kernel here")


# speedup vs baseline: 24.4123x; 24.4123x over previous
import functools
from functools import partial

import jax
import jax.numpy as jnp
from jax import lax
from jax.experimental import pallas as pl
from jax.experimental.pallas import tpu as pltpu
from jax.experimental.pallas import tpu_sc as plsc

F32 = jnp.float32
I32 = jnp.int32
NSUB = 16
NC = 2
KB = 128
ZCH = 112


def _ceil_to(x, m):
    return (x + m - 1) // m * m


def _dims(n_nodes, n_edges):
    nrows = _ceil_to((n_nodes + 1 + NSUB - 1) // NSUB, ZCH)
    npad = NSUB * nrows
    esub32 = _ceil_to((n_edges + NC * NSUB - 1) // (NC * NSUB), KB)
    epad = NC * NSUB * esub32
    return npad, nrows, epad, esub32


def _mesh():
    return plsc.VectorSubcoreMesh(core_axis_name="c", subcore_axis_name="s")


def _k_geo_body(n, npad, nrows, epad, esub,
                posp, srcp, dstp, ps_out, pd_out, idt_out,
                sidx, didx, psb, pdb, ones, zb, ivb, ivrep, sem, dacc):
    cid = lax.axis_index("c")
    sid = lax.axis_index("s")
    nb = esub // KB
    half = nb // NC

    def fill(i, _):
        zb[pl.ds(i * 16, 16)] = jnp.zeros((16,), F32)
        return 0
    lax.fori_loop(0, nrows // 16, fill, 0)

    def fill1(i, _):
        ones[pl.ds(i * 16, 16)] = jnp.ones((16,), F32)
        return 0
    lax.fori_loop(0, KB // 16, fill1, 0)

    pltpu.sync_copy(zb, dacc.at[pl.ds(sid * nrows, nrows)])
    plsc.subcore_barrier()

    def batch(b, _):
        e0 = sid * esub + b * KB
        pltpu.sync_copy(dstp.at[pl.ds(e0, KB)], didx)
        pltpu.sync_copy(ones, dacc.at[didx], add=True)

        lo = cid * half

        @pl.when(jnp.logical_and(b >= lo, b < lo + half))
        def _():
            pltpu.sync_copy(srcp.at[pl.ds(e0, KB)], sidx)
            pltpu.async_copy(posp.at[sidx], psb, sem).wait()
            pltpu.sync_copy(psb, ps_out.at[pl.ds(e0, KB)])
            pltpu.async_copy(posp.at[didx], pdb, sem).wait()
            pltpu.sync_copy(pdb, pd_out.at[pl.ds(e0, KB)])
        return 0
    lax.fori_loop(0, nb, batch, 0)
    plsc.subcore_barrier()

    pltpu.sync_copy(dacc.at[pl.ds(sid * nrows, nrows)], ivb)

    def inv(i, _):
        v = 1.0 / jnp.maximum(ivb[pl.ds(i * 16, 16)], 1.0)
        for j in range(16):
            ivrep[i * 16 + j, pl.ds(0, 16)] = jnp.full((16,), v[j], F32)
        return 0
    lax.fori_loop(0, nrows // 16, inv, 0)

    @pl.when(cid == 0)
    def _():
        pltpu.sync_copy(ivrep, idt_out.at[pl.ds(sid * nrows, nrows)])


def _k_geo(n, npad, nrows, epad, esub, posp, srcp, dstp):
    body = partial(_k_geo_body, n, npad, nrows, epad, esub)
    return pl.kernel(
        body,
        out_type=[
            jax.ShapeDtypeStruct((epad, 8), F32),
            jax.ShapeDtypeStruct((epad, 8), F32),
            jax.ShapeDtypeStruct((npad, 16), F32),
        ],
        mesh=_mesh(),
        compiler_params=pltpu.CompilerParams(use_tc_tiling_on_sc=False),
        scratch_types=[
            pltpu.VMEM((KB,), I32),
            pltpu.VMEM((KB,), I32),
            pltpu.VMEM((KB, 8), F32),
            pltpu.VMEM((KB, 8), F32),
            pltpu.VMEM((KB,), F32),
            pltpu.VMEM((nrows,), F32),
            pltpu.VMEM((nrows,), F32),
            pltpu.VMEM((nrows, 16), F32),
            pltpu.SemaphoreType.DMA,
            pltpu.VMEM_SHARED((npad,), F32),
        ],
    )(posp, srcp, dstp)


def _k_edge_body(npad, nrows, epad, esub,
                 srcp, dstp, gg, rh, idt, ta, tb, out,
                 sidx, didx, gb, rhb, tbuf, ideb, mb, zb, sem, acc):
    cid = lax.axis_index("c")
    sid = lax.axis_index("s")
    nb = esub // KB

    def fillz(i, _):
        zb[i, pl.ds(0, 16)] = jnp.zeros((16,), F32)
        zb[i, pl.ds(16, 16)] = jnp.zeros((16,), F32)
        return 0
    lax.fori_loop(0, ZCH, fillz, 0)

    def init(j, _):
        pltpu.sync_copy(zb, acc.at[pl.ds(sid * nrows + j * ZCH, ZCH)])
        return 0
    lax.fori_loop(0, nrows // ZCH, init, 0)
    plsc.subcore_barrier()

    def stage(b, table):
        e0 = sid * esub + b * KB
        pltpu.sync_copy(srcp.at[pl.ds(e0, KB)], sidx)
        pltpu.sync_copy(dstp.at[pl.ds(e0, KB)], didx)
        pltpu.sync_copy(gg.at[pl.ds(e0, KB)], gb)
        pltpu.sync_copy(rh.at[pl.ds(e0 * 4, KB * 4)], rhb)
        pltpu.async_copy(table.at[sidx], tbuf, sem).wait()
        pltpu.async_copy(idt.at[didx], ideb, sem).wait()

    def batch_a(b, _):
        stage(b, ta)

        def group(g, _):
            rv = [rhb[pl.ds(g * 64 + 16 * t, 16)] for t in range(4)]
            for j in range(16):
                k = g * 16 + j
                g1 = gb[k, pl.ds(16, 16)]
                g2 = gb[k, pl.ds(32, 16)]
                vx = tbuf[k, pl.ds(0, 16)]
                vy = tbuf[k, pl.ds(16, 16)]
                bb = tbuf[k, pl.ds(32, 16)]
                d = ideb[k, pl.ds(0, 16)]
                o = (j % 4) * 4
                rx = rv[j // 4][o]
                ry = rv[j // 4][o + 1]
                q = g2 * bb
                mb[k, pl.ds(0, 16)] = d * (g1 * vx + q * rx)
                mb[k, pl.ds(16, 16)] = d * (g1 * vy + q * ry)
            return 0
        lax.fori_loop(0, KB // 16, group, 0)
        pltpu.sync_copy(mb, acc.at[didx], add=True)
        return 0

    def batch_b(b, _):
        stage(b, tb)

        def group(g, _):
            rv = [rhb[pl.ds(g * 64 + 16 * t, 16)] for t in range(4)]
            for j in range(16):
                k = g * 16 + j
                g0 = gb[k, pl.ds(0, 16)]
                g1 = gb[k, pl.ds(16, 16)]
                g2 = gb[k, pl.ds(32, 16)]
                aa = tbuf[k, pl.ds(0, 16)]
                vz = tbuf[k, pl.ds(16, 16)]
                bb = tbuf[k, pl.ds(32, 16)]
                d = ideb[k, pl.ds(0, 16)]
                rz = rv[j // 4][(j % 4) * 4 + 2]
                mb[k, pl.ds(0, 16)] = d * (g0 * aa)
                mb[k, pl.ds(16, 16)] = d * (g1 * vz + (g2 * bb) * rz)
            return 0
        lax.fori_loop(0, KB // 16, group, 0)
        pltpu.sync_copy(mb, acc.at[didx], add=True)
        return 0

    @pl.when(cid == 0)
    def _():
        lax.fori_loop(0, nb, batch_a, 0)

    @pl.when(cid == 1)
    def _():
        lax.fori_loop(0, nb, batch_b, 0)

    plsc.subcore_barrier()
    pltpu.sync_copy(acc.at[pl.ds(sid * nrows, nrows)],
                    out.at[cid, pl.ds(sid * nrows, nrows)])


def _k_edge(npad, nrows, epad, esub, srcp, dstp, gg, rh, idt, ta, tb):
    body = partial(_k_edge_body, npad, nrows, epad, esub)
    return pl.kernel(
        body,
        out_type=jax.ShapeDtypeStruct((NC, npad, 32), F32),
        mesh=_mesh(),
        compiler_params=pltpu.CompilerParams(use_tc_tiling_on_sc=False),
        scratch_types=[
            pltpu.VMEM((KB,), I32),
            pltpu.VMEM((KB,), I32),
            pltpu.VMEM((KB, 48), F32),
            pltpu.VMEM((KB * 4,), F32),
            pltpu.VMEM((KB, 48), F32),
            pltpu.VMEM((KB, 16), F32),
            pltpu.VMEM((KB, 32), F32),
            pltpu.VMEM((ZCH, 32), F32),
            pltpu.SemaphoreType.DMA,
            pltpu.VMEM_SHARED((npad, 32), F32),
        ],
    )(srcp, dstp, gg, rh, idt, ta, tb)


def _k_sum_body(npad, epad, esub,
                srcp, dstp, gg, rh, idt, tab, out,
                sidx, didx, gb, rhb, tbuf, ideb, ob, sem):
    cid = lax.axis_index("c")
    sid = lax.axis_index("s")
    wid = sid * NC + cid
    nb = esub // KB

    def batch(b, carry):
        e0 = wid * esub + b * KB
        pltpu.sync_copy(srcp.at[pl.ds(e0, KB)], sidx)
        pltpu.sync_copy(dstp.at[pl.ds(e0, KB)], didx)
        pltpu.sync_copy(gg.at[pl.ds(e0, KB)], gb)
        pltpu.sync_copy(rh.at[pl.ds(e0 * 4, KB * 4)], rhb)
        pltpu.async_copy(tab.at[sidx], tbuf, sem).wait()
        pltpu.async_copy(idt.at[didx], ideb, sem).wait()

        def group(g, c):
            (s0l, s0h, sxl, sxh, syl, syh, szl, szh) = c
            rv = [rhb[pl.ds(g * 64 + 16 * t, 16)] for t in range(4)]
            for j in range(16):
                k = g * 16 + j
                d = ideb[k, pl.ds(0, 16)]
                o = (j % 4) * 4
                rx = rv[j // 4][o]
                ry = rv[j // 4][o + 1]
                rz = rv[j // 4][o + 2]
                g0l = gb[k, pl.ds(0, 16)] * d
                g0h = gb[k, pl.ds(16, 16)] * d
                g1l = gb[k, pl.ds(32, 16)] * d
                g1h = gb[k, pl.ds(48, 16)] * d
                g2l = gb[k, pl.ds(64, 16)] * d
                g2h = gb[k, pl.ds(80, 16)] * d
                ql = g2l * tbuf[k, pl.ds(32, 16)]
                qh = g2h * tbuf[k, pl.ds(48, 16)]
                s0l = s0l + g0l * tbuf[k, pl.ds(0, 16)]
                s0h = s0h + g0h * tbuf[k, pl.ds(16, 16)]
                sxl = sxl + g1l * tbuf[k, pl.ds(64, 16)] + ql * rx
                sxh = sxh + g1h * tbuf[k, pl.ds(80, 16)] + qh * rx
                syl = syl + g1l * tbuf[k, pl.ds(96, 16)] + ql * ry
                syh = syh + g1h * tbuf[k, pl.ds(112, 16)] + qh * ry
                szl = szl + g1l * tbuf[k, pl.ds(128, 16)] + ql * rz
                szh = szh + g1h * tbuf[k, pl.ds(144, 16)] + qh * rz
            return (s0l, s0h, sxl, sxh, syl, syh, szl, szh)
        return lax.fori_loop(0, KB // 16, group, carry)

    z = jnp.zeros((16,), F32)
    acc = lax.fori_loop(0, nb, batch, (z,) * 8)
    for i, v in enumerate(acc):
        ob[pl.ds(i * 16, 16)] = v
    pltpu.sync_copy(ob, out.at[wid])


def _k_sum(npad, epad, esub, srcp, dstp, gg, rh, idt, tab):
    body = partial(_k_sum_body, npad, epad, esub)
    return pl.kernel(
        body,
        out_type=jax.ShapeDtypeStruct((NC * NSUB, 128), F32),
        mesh=_mesh(),
        compiler_params=pltpu.CompilerParams(use_tc_tiling_on_sc=False),
        scratch_types=[
            pltpu.VMEM((KB,), I32),
            pltpu.VMEM((KB,), I32),
            pltpu.VMEM((KB, 96), F32),
            pltpu.VMEM((KB * 4,), F32),
            pltpu.VMEM((KB, 160), F32),
            pltpu.VMEM((KB, 16), F32),
            pltpu.VMEM((128,), F32),
            pltpu.SemaphoreType.DMA,
        ],
    )(srcp, dstp, gg, rh, idt, tab)


def _t_rad_body(n_edges, wr1, br1, wr2a, br2a, wr2c, br2c,
                ps, pd, ea, rh, g0, g1, g2):
    i = pl.program_id(0)
    be = ps.shape[0]
    p_s = ps[...]
    p_d = pd[...]
    r = p_d[:, 0:3] - p_s[:, 0:3]
    d2 = jnp.sum(r * r, axis=1, keepdims=True) + 1e-8
    dist = jnp.sqrt(d2)
    rhat = r / dist
    rh[...] = jnp.concatenate([rhat, jnp.zeros((be, 1), F32)], axis=1)
    eidx = i * be + lax.broadcasted_iota(I32, (be, 1), 0)
    valid = (eidx < n_edges).astype(F32)
    ef = jnp.concatenate([dist, ea[...]], axis=1)
    wr1a, br1a = wr1[...], br1[...]
    wr2aa, br2aa = wr2a[...], br2a[...]
    for l, out in ((0, g0), (1, g1), (2, g2)):
        h = jnp.maximum(jnp.dot(ef, wr1a[l], preferred_element_type=F32)
                        + br1a[l][None, :], 0.0)
        if l < 2:
            g = jnp.dot(h, wr2aa[l], preferred_element_type=F32) + br2aa[l][None, :]
        else:
            g = jnp.dot(h, wr2c[...], preferred_element_type=F32) + br2c[...][None, :]
        out[...] = g * valid


def _t_rad(n_edges, epad, conv, ps, pd, eap):
    be = 2048
    grid = epad // be
    wr1 = jnp.stack([c["Wr1"] for c in conv])
    br1 = jnp.stack([c["br1"] for c in conv])
    wr2a = jnp.stack([conv[0]["Wr2"], conv[1]["Wr2"]])
    br2a = jnp.stack([conv[0]["br2"], conv[1]["br2"]])
    wr2c = conv[2]["Wr2"]
    br2c = conv[2]["br2"]
    full = lambda a: pl.BlockSpec(a.shape, lambda i: (0,) * a.ndim)
    eb = lambda w: pl.BlockSpec((be, w), lambda i: (i, 0))
    return pl.pallas_call(
        partial(_t_rad_body, n_edges),
        grid=(grid,),
        in_specs=[full(wr1), full(br1), full(wr2a), full(br2a), full(wr2c),
                  full(br2c), eb(8), eb(8), eb(4)],
        out_specs=[eb(4), eb(48), eb(48), eb(96)],
        out_shape=[
            jax.ShapeDtypeStruct((epad, 4), F32),
            jax.ShapeDtypeStruct((epad, 48), F32),
            jax.ShapeDtypeStruct((epad, 48), F32),
            jax.ShapeDtypeStruct((epad, 96), F32),
        ],
    )(wr1, br1, wr2a, br2a, wr2c, br2c, ps, pd, eap)


def _t_tab0_body(w0, mv, ta, tb):
    w = w0[...]
    m = mv[...][:, 0:1]
    a = m * w[0][None, :]
    b = m * w[2][None, :]
    vx = mv[...][:, 1:2] * w[1][None, :]
    vy = mv[...][:, 2:3] * w[1][None, :]
    vz = mv[...][:, 3:4] * w[1][None, :]
    ta[...] = jnp.concatenate([vx, vy, b], axis=1)
    tb[...] = jnp.concatenate([a, vz, b], axis=1)


def _t_tab0(npad, bn, conv0, mv):
    grid = npad // bn
    w0 = jnp.stack([conv0["Ws"][0], conv0["Wv"][0], conv0["Wsv"][0]])
    return pl.pallas_call(
        _t_tab0_body,
        grid=(grid,),
        in_specs=[pl.BlockSpec(w0.shape, lambda i: (0, 0)),
                  pl.BlockSpec((bn, 4), lambda i: (i, 0))],
        out_specs=[pl.BlockSpec((bn, 48), lambda i: (i, 0))] * 2,
        out_shape=[jax.ShapeDtypeStruct((npad, 48), F32)] * 2,
    )(w0, mv)


def _t_node0_body(u0, wn, bn_, w1, mv, out0, ta, tb, h0o, h1o):
    o = out0[...]
    u = u0[...]
    w = w1[...]
    amx, amy = o[0, :, 0:16], o[0, :, 16:32]
    am0, amz = o[1, :, 0:16], o[1, :, 16:32]
    m = mv[...][:, 0:1]
    h0 = jnp.maximum(am0 + m * u[0][None, :], 0.0)
    h1x = amx + mv[...][:, 1:2] * u[1][None, :]
    h1y = amy + mv[...][:, 2:3] * u[1][None, :]
    h1z = amz + mv[...][:, 3:4] * u[1][None, :]
    nrm = jnp.sqrt(h1x * h1x + h1y * h1y + h1z * h1z + 1e-12)
    sc = jax.nn.sigmoid(jnp.dot(nrm, wn[...], preferred_element_type=F32)
                        + bn_[...])
    h1x, h1y, h1z = h1x * sc, h1y * sc, h1z * sc
    a = jnp.dot(h0, w[0], preferred_element_type=F32)
    b = jnp.dot(h0, w[1], preferred_element_type=F32)
    vx = jnp.dot(h1x, w[2], preferred_element_type=F32)
    vy = jnp.dot(h1y, w[2], preferred_element_type=F32)
    vz = jnp.dot(h1z, w[2], preferred_element_type=F32)
    ta[...] = jnp.concatenate([vx, vy, b], axis=1)
    tb[...] = jnp.concatenate([a, vz, b], axis=1)
    h0o[...] = h0
    h1o[...] = jnp.concatenate([h1x, h1y, h1z], axis=1)


def _t_node0(npad, bn, conv0, norm0, conv1, mv, out0):
    grid = npad // bn
    u0 = jnp.stack([conv0["Us"][0], conv0["Uv"][0]])
    w1 = jnp.stack([conv1["Ws"], conv1["Wsv"], conv1["Wv"]])
    bn1 = norm0["bn"][None, :]
    full = lambda a: pl.BlockSpec(a.shape, lambda i: (0,) * a.ndim)
    return pl.pallas_call(
        _t_node0_body,
        grid=(grid,),
        in_specs=[full(u0), full(norm0["Wn"]), full(bn1), full(w1),
                  pl.BlockSpec((bn, 4), lambda i: (i, 0)),
                  pl.BlockSpec((NC, bn, 32), lambda i: (0, i, 0))],
        out_specs=[pl.BlockSpec((bn, 48), lambda i: (i, 0)),
                   pl.BlockSpec((bn, 48), lambda i: (i, 0)),
                   pl.BlockSpec((bn, 16), lambda i: (i, 0)),
                   pl.BlockSpec((bn, 48), lambda i: (i, 0))],
        out_shape=[jax.ShapeDtypeStruct((npad, 48), F32),
                   jax.ShapeDtypeStruct((npad, 48), F32),
                   jax.ShapeDtypeStruct((npad, 16), F32),
                   jax.ShapeDtypeStruct((npad, 48), F32)],
    )(u0, norm0["Wn"], bn1, w1, mv, out0)


def _t_node1_body(u1, wn, bn_, w2, h0i, h1i, out1, tab, sh0, sh1):
    i = pl.program_id(0)
    o = out1[...]
    u = u1[...]
    w = w2[...]
    amx, amy = o[0, :, 0:16], o[0, :, 16:32]
    am0, amz = o[1, :, 0:16], o[1, :, 16:32]
    h1p = h1i[...]
    h0 = jnp.maximum(am0 + jnp.dot(h0i[...], u[0], preferred_element_type=F32), 0.0)
    h1x = amx + jnp.dot(h1p[:, 0:16], u[1], preferred_element_type=F32)
    h1y = amy + jnp.dot(h1p[:, 16:32], u[1], preferred_element_type=F32)
    h1z = amz + jnp.dot(h1p[:, 32:48], u[1], preferred_element_type=F32)
    nrm = jnp.sqrt(h1x * h1x + h1y * h1y + h1z * h1z + 1e-12)
    sc = jax.nn.sigmoid(jnp.dot(nrm, wn[...], preferred_element_type=F32)
                        + bn_[...])
    h1x, h1y, h1z = h1x * sc, h1y * sc, h1z * sc
    a = jnp.dot(h0, w[0], preferred_element_type=F32)
    b = jnp.dot(h0, w[1], preferred_element_type=F32)
    vx = jnp.dot(h1x, w[2], preferred_element_type=F32)
    vy = jnp.dot(h1y, w[2], preferred_element_type=F32)
    vz = jnp.dot(h1z, w[2], preferred_element_type=F32)
    tab[...] = jnp.concatenate([a, b, vx, vy, vz], axis=1)

    @pl.when(i == 0)
    def _():
        sh0[...] = jnp.zeros_like(sh0)
        sh1[...] = jnp.zeros_like(sh1)
    sh0[...] += jnp.sum(h0, axis=0, keepdims=True)
    sh1[...] += jnp.concatenate(
        [jnp.sum(h1x, axis=0, keepdims=True),
         jnp.sum(h1y, axis=0, keepdims=True),
         jnp.sum(h1z, axis=0, keepdims=True)], axis=1)


def _t_node1(npad, bn, conv1, norm1, conv2, h0, h1, out1):
    grid = npad // bn
    u1 = jnp.stack([conv1["Us"], conv1["Uv"]])
    w2 = jnp.stack([conv2["Ws"], conv2["Wsv"], conv2["Wv"]])
    bn1 = norm1["bn"][None, :]
    full = lambda a: pl.BlockSpec(a.shape, lambda i: (0,) * a.ndim)
    return pl.pallas_call(
        _t_node1_body,
        grid=(grid,),
        in_specs=[full(u1), full(norm1["Wn"]), full(bn1), full(w2),
                  pl.BlockSpec((bn, 16), lambda i: (i, 0)),
                  pl.BlockSpec((bn, 48), lambda i: (i, 0)),
                  pl.BlockSpec((NC, bn, 32), lambda i: (0, i, 0))],
        out_specs=[pl.BlockSpec((bn, 160), lambda i: (i, 0)),
                   pl.BlockSpec((1, 16), lambda i: (0, 0)),
                   pl.BlockSpec((1, 48), lambda i: (0, 0))],
        out_shape=[jax.ShapeDtypeStruct((npad, 160), F32),
                   jax.ShapeDtypeStruct((1, 16), F32),
                   jax.ShapeDtypeStruct((1, 48), F32)],
    )(u1, norm1["Wn"], bn1, w2, h0, h1, out1)


def _t_fin_body(n_nodes, u2, f0, f1, o2, sh0, sh1, o0, o1):
    s = jnp.sum(o2[...], axis=0)
    u, fa, fb = u2[...], f0[...], f1[...]
    inv_n = 1.0 / n_nodes
    mh0 = (s[0:32][None, :]
           + jnp.dot(sh0[...], u[0], preferred_element_type=F32)) * inv_n
    o0[...] = jnp.dot(jnp.dot(mh0, fa[0], preferred_element_type=F32),
                      fb[0], preferred_element_type=F32)
    rows = []
    for i in range(3):
        mh1 = (s[32 + 32 * i:64 + 32 * i][None, :]
               + jnp.dot(sh1[...][:, 16 * i:16 * i + 16], u[1],
                         preferred_element_type=F32)) * inv_n
        rows.append(jnp.dot(jnp.dot(mh1, fa[1], preferred_element_type=F32),
                            fb[1], preferred_element_type=F32))
    o1[...] = jnp.concatenate(rows, axis=0)


def _t_fin(n_nodes, conv2, fc, o2, sh0, sh1):
    u2 = jnp.stack([conv2["Us"], conv2["Uv"]])
    f0 = jnp.stack([fc["Wf0_s"], fc["Wf0_v"]])
    f1 = jnp.stack([fc["Wf1_s"], fc["Wf1_v"]])
    full = lambda a: pl.BlockSpec(a.shape, lambda: (0,) * a.ndim)
    return pl.pallas_call(
        partial(_t_fin_body, float(n_nodes)),
        in_specs=[full(u2), full(f0), full(f1), full(o2), full(sh0), full(sh1)],
        out_specs=[full(jnp.zeros((1, 2))), full(jnp.zeros((3, 2)))],
        out_shape=[jax.ShapeDtypeStruct((1, 2), F32),
                   jax.ShapeDtypeStruct((3, 2), F32)],
    )(u2, f0, f1, o2, sh0, sh1)


def kernel(pos, mass, velocity, edge_attr, edge_index, params):
    n_nodes = pos.shape[0]
    n_edges = edge_index.shape[1]
    npad, nrows, epad, esub32 = _dims(n_nodes, n_edges)
    esub16 = esub32 * NC
    bn = nrows

    conv = params["conv"]
    norm = params["norm"]
    fc = params["fc"]

    src = edge_index[0].astype(I32)
    dst = edge_index[1].astype(I32)
    pad_e = epad - n_edges
    srcp = jnp.concatenate([src, jnp.full((pad_e,), n_nodes, I32)])
    dstp = jnp.concatenate([dst, jnp.full((pad_e,), n_nodes, I32)])
    posp = jnp.zeros((npad, 8), F32).at[:n_nodes, 0:3].set(pos)
    mv = jnp.zeros((npad, 4), F32).at[:n_nodes, 0:1].set(mass)
    mv = mv.at[:n_nodes, 1:4].set(velocity)
    eap = jnp.zeros((epad, 4), F32).at[:n_edges].set(edge_attr)

    ps, pd, idt = _k_geo(n_nodes, npad, nrows, epad, esub16, posp, srcp, dstp)
    rh, g0, g1, g2 = _t_rad(n_edges, epad, conv, ps, pd, eap)
    rhf = rh.reshape(-1)
    ta0, tb0 = _t_tab0(npad, bn, conv[0], mv)
    out0 = _k_edge(npad, nrows, epad, esub16, srcp, dstp, g0, rhf, idt, ta0, tb0)
    ta1, tb1, h0, h1 = _t_node0(npad, bn, conv[0], norm[0], conv[1], mv, out0)
    out1 = _k_edge(npad, nrows, epad, esub16, srcp, dstp, g1, rhf, idt, ta1, tb1)
    tab, sh0, sh1 = _t_node1(npad, bn, conv[1], norm[1], conv[2], h0, h1, out1)
    o2 = _k_sum(npad, epad, esub32, srcp, dstp, g2, rhf, idt, tab)
    o0, o1 = _t_fin(n_nodes, conv[2], fc, o2, sh0, sh1)
    return o0, jnp.transpose(o1)[None]


# speedup vs baseline: 35.1463x; 1.4397x over previous
import functools
from functools import partial

import jax
import jax.numpy as jnp
from jax import lax
from jax.experimental import pallas as pl
from jax.experimental.pallas import tpu as pltpu
from jax.experimental.pallas import tpu_sc as plsc

F32 = jnp.float32
I32 = jnp.int32
NSUB = 16
NC = 2
KB = 128
ZCH = 112
SB = 256
SUB = 2


def _ceil_to(x, m):
    return (x + m - 1) // m * m


def _dims(n_nodes, n_edges):
    nrows = _ceil_to((n_nodes + 1 + NSUB - 1) // NSUB, ZCH)
    npad = NSUB * nrows
    esub32 = _ceil_to((n_edges + NC * NSUB - 1) // (NC * NSUB), SB)
    epad = NC * NSUB * esub32
    return npad, nrows, epad, esub32


def _mesh():
    return plsc.VectorSubcoreMesh(core_axis_name="c", subcore_axis_name="s")


def _k_geo_body(n, npad, nrows, epad, esub,
                posp, srcp, dstp, ps_out, pd_out, idt_out,
                sidx, didx, psb, pdb, ones, zb, ivb, ivrep, sem, dacc):
    cid = lax.axis_index("c")
    sid = lax.axis_index("s")
    nb = esub // KB
    half = nb // NC

    def fill(i, _):
        zb[pl.ds(i * 16, 16)] = jnp.zeros((16,), F32)
        return 0
    lax.fori_loop(0, nrows // 16, fill, 0)

    def fill1(i, _):
        ones[pl.ds(i * 16, 16)] = jnp.ones((16,), F32)
        return 0
    lax.fori_loop(0, KB // 16, fill1, 0)

    pltpu.sync_copy(zb, dacc.at[pl.ds(sid * nrows, nrows)])
    plsc.subcore_barrier()

    def batch(b, _):
        e0 = sid * esub + b * KB
        pltpu.sync_copy(dstp.at[pl.ds(e0, KB)], didx)
        pltpu.sync_copy(ones, dacc.at[didx], add=True)

        lo = cid * half

        @pl.when(jnp.logical_and(b >= lo, b < lo + half))
        def _():
            pltpu.sync_copy(srcp.at[pl.ds(e0, KB)], sidx)
            pltpu.async_copy(posp.at[sidx], psb, sem).wait()
            pltpu.sync_copy(psb, ps_out.at[pl.ds(e0, KB)])
            pltpu.async_copy(posp.at[didx], pdb, sem).wait()
            pltpu.sync_copy(pdb, pd_out.at[pl.ds(e0, KB)])
        return 0
    lax.fori_loop(0, nb, batch, 0)
    plsc.subcore_barrier()

    pltpu.sync_copy(dacc.at[pl.ds(sid * nrows, nrows)], ivb)

    def inv(i, _):
        v = 1.0 / jnp.maximum(ivb[pl.ds(i * 16, 16)], 1.0)
        for j in range(16):
            ivrep[i * 16 + j, pl.ds(0, 16)] = jnp.full((16,), v[j], F32)
        return 0
    lax.fori_loop(0, nrows // 16, inv, 0)

    @pl.when(cid == 0)
    def _():
        pltpu.sync_copy(ivrep, idt_out.at[pl.ds(sid * nrows, nrows)])


def _k_geo(n, npad, nrows, epad, esub, posp, srcp, dstp):
    body = partial(_k_geo_body, n, npad, nrows, epad, esub)
    return pl.kernel(
        body,
        out_type=[
            jax.ShapeDtypeStruct((epad, 8), F32),
            jax.ShapeDtypeStruct((epad, 8), F32),
            jax.ShapeDtypeStruct((npad, 16), F32),
        ],
        mesh=_mesh(),
        compiler_params=pltpu.CompilerParams(use_tc_tiling_on_sc=False),
        scratch_types=[
            pltpu.VMEM((KB,), I32),
            pltpu.VMEM((KB,), I32),
            pltpu.VMEM((KB, 8), F32),
            pltpu.VMEM((KB, 8), F32),
            pltpu.VMEM((KB,), F32),
            pltpu.VMEM((nrows,), F32),
            pltpu.VMEM((nrows,), F32),
            pltpu.VMEM((nrows, 16), F32),
            pltpu.SemaphoreType.DMA,
            pltpu.VMEM_SHARED((npad,), F32),
        ],
    )(posp, srcp, dstp)


def _k_edge_body(npad, nrows, epad, esub,
                 src2, dst2, gga, ggb, ta, tb, out,
                 sidx, didx, gbuf, tbuf, mb,
                 si0, si1, sd0, sd1, sg0, sg1, acc):
    cid = lax.axis_index("c")
    sid = lax.axis_index("s")
    ng = esub // KB
    rsub = esub // KB

    def fillz(i, _):
        mb[i, pl.ds(0, 16)] = jnp.zeros((16,), F32)
        mb[i, pl.ds(16, 16)] = jnp.zeros((16,), F32)
        return 0
    lax.fori_loop(0, KB, fillz, 0)

    def init(j, _):
        pltpu.sync_copy(mb, acc.at[pl.ds(sid * nrows + j * KB, KB)])
        return 0
    lax.fori_loop(0, nrows // KB, init, 0)

    if nrows % KB:
        pltpu.sync_copy(mb.at[pl.ds(0, nrows % KB)],
                        acc.at[pl.ds(sid * nrows + (nrows // KB) * KB,
                                     nrows % KB)])
    plsc.subcore_barrier()

    sis = (si0, si1)
    sds = (sd0, sd1)
    sgs = (sg0, sg1)

    def issue_sidx(g, s):
        pltpu.async_copy(src2.at[pl.ds(sid * rsub + g, 1)], sidx.at[s], sis[s])

    def wait_sidx(s):
        pltpu.make_async_copy(src2.at[pl.ds(0, 1)], sidx.at[s], sis[s]).wait()

    def issue_didx(g, s):
        pltpu.async_copy(dst2.at[pl.ds(sid * rsub + g, 1)], didx.at[s], sds[s])

    def wait_didx(s):
        pltpu.make_async_copy(dst2.at[pl.ds(0, 1)], didx.at[s], sds[s]).wait()

    def run(table, gsrc, compute16):
        def issue_main(g, s):
            pltpu.async_copy(gsrc.at[pl.ds(sid * esub + g * KB, KB)],
                             gbuf.at[s], sgs[s])
            pltpu.async_copy(table.at[sidx.at[s, 0]], tbuf.at[s], sgs[s])

        def wait_main(s):
            pltpu.make_async_copy(gsrc.at[pl.ds(0, KB)], gbuf.at[s],
                                  sgs[s]).wait()
            pltpu.make_async_copy(table.at[pl.ds(0, KB)], tbuf.at[s],
                                  sgs[s]).wait()

        def compute(g, s):
            def grp(q, _):
                for j in range(16):
                    compute16(s, q * 16 + j)
                return 0
            lax.fori_loop(0, KB // 16, grp, 0)
            pltpu.sync_copy(mb, acc.at[didx.at[s, 0]], add=True)

        issue_sidx(0, 0)
        issue_didx(0, 0)
        wait_sidx(0)
        issue_main(0, 0)
        issue_sidx(1, 1)
        issue_didx(1, 1)

        def half(g, s, n):
            @pl.when(g < ng - 1)
            def _():
                wait_sidx(n)
            wait_main(s)

            @pl.when(g < ng - 1)
            def _():
                issue_main(g + 1, n)

            @pl.when(g < ng - 2)
            def _():
                issue_sidx(g + 2, s)
            wait_didx(s)
            compute(g, s)

            @pl.when(g < ng - 2)
            def _():
                issue_didx(g + 2, s)

        def body(i, _):
            half(2 * i, 0, 1)
            half(2 * i + 1, 1, 0)
            return 0
        lax.fori_loop(0, ng // 2, body, 0)

    def comp_a(s, k):
        g1 = gbuf[s, k, pl.ds(0, 16)]
        g2x = gbuf[s, k, pl.ds(16, 16)]
        g2y = gbuf[s, k, pl.ds(32, 16)]
        vx = tbuf[s, k, pl.ds(0, 16)]
        vy = tbuf[s, k, pl.ds(16, 16)]
        bb = tbuf[s, k, pl.ds(32, 16)]
        mb[k, pl.ds(0, 16)] = g1 * vx + g2x * bb
        mb[k, pl.ds(16, 16)] = g1 * vy + g2y * bb

    def comp_b(s, k):
        g0 = gbuf[s, k, pl.ds(0, 16)]
        g1 = gbuf[s, k, pl.ds(16, 16)]
        g2z = gbuf[s, k, pl.ds(32, 16)]
        aa = tbuf[s, k, pl.ds(0, 16)]
        vz = tbuf[s, k, pl.ds(16, 16)]
        bb = tbuf[s, k, pl.ds(32, 16)]
        mb[k, pl.ds(0, 16)] = g0 * aa
        mb[k, pl.ds(16, 16)] = g1 * vz + g2z * bb

    @pl.when(cid == 0)
    def _():
        run(ta, gga, comp_a)

    @pl.when(cid == 1)
    def _():
        run(tb, ggb, comp_b)

    plsc.subcore_barrier()
    pltpu.sync_copy(acc.at[pl.ds(sid * nrows, nrows)],
                    out.at[cid, pl.ds(sid * nrows, nrows)])


def _k_edge(npad, nrows, epad, esub, src2, dst2, gga, ggb, ta, tb):
    body = partial(_k_edge_body, npad, nrows, epad, esub)
    return pl.kernel(
        body,
        out_type=jax.ShapeDtypeStruct((NC, npad, 32), F32),
        mesh=_mesh(),
        compiler_params=pltpu.CompilerParams(use_tc_tiling_on_sc=False),
        scratch_types=[
            pltpu.VMEM((2, 1, KB), I32),
            pltpu.VMEM((2, 1, KB), I32),
            pltpu.VMEM((2, KB, 48), F32),
            pltpu.VMEM((2, KB, 48), F32),
            pltpu.VMEM((KB, 32), F32),
            pltpu.SemaphoreType.DMA,
            pltpu.SemaphoreType.DMA,
            pltpu.SemaphoreType.DMA,
            pltpu.SemaphoreType.DMA,
            pltpu.SemaphoreType.DMA,
            pltpu.SemaphoreType.DMA,
            pltpu.VMEM_SHARED((npad, 32), F32),
        ],
    )(src2, dst2, gga, ggb, ta, tb)


def _k_sum_body(npad, epad, esub,
                src2, dst2, gg, idt, tab, out,
                sidx, didx, gbuf, tbuf, ideb, ob,
                si0, si1, sg0, sg1):
    cid = lax.axis_index("c")
    sid = lax.axis_index("s")
    wid = sid * NC + cid
    ng = esub // 128
    rsub = esub // 128

    sis = (si0, si1)
    sgs = (sg0, sg1)

    def issue_idx(g, s):
        pltpu.async_copy(src2.at[pl.ds(wid * rsub + g, 1)], sidx.at[s], sis[s])
        pltpu.async_copy(dst2.at[pl.ds(wid * rsub + g, 1)], didx.at[s], sis[s])

    def wait_idx(s):
        pltpu.make_async_copy(src2.at[pl.ds(0, 1)], sidx.at[s], sis[s]).wait()
        pltpu.make_async_copy(dst2.at[pl.ds(0, 1)], didx.at[s], sis[s]).wait()

    def issue_main(g, s):
        pltpu.async_copy(gg.at[pl.ds(wid * esub + g * 128, 128)],
                         gbuf.at[s], sgs[s])
        pltpu.async_copy(tab.at[sidx.at[s, 0]], tbuf.at[s], sgs[s])
        pltpu.async_copy(idt.at[didx.at[s, 0]], ideb.at[s], sgs[s])

    def wait_main(s):
        pltpu.make_async_copy(gg.at[pl.ds(0, 128)], gbuf.at[s], sgs[s]).wait()
        pltpu.make_async_copy(tab.at[pl.ds(0, 128)], tbuf.at[s], sgs[s]).wait()
        pltpu.make_async_copy(idt.at[pl.ds(0, 128)], ideb.at[s], sgs[s]).wait()

    def compute(s, carry):
        def grp(q, c):
            (s0l, s0h, sxl, sxh, syl, syh, szl, szh) = c
            for j in range(16):
                k = q * 16 + j
                d = ideb[s, k, pl.ds(0, 16)]
                al = tbuf[s, k, pl.ds(0, 16)] * d
                ah = tbuf[s, k, pl.ds(16, 16)] * d
                bl = tbuf[s, k, pl.ds(32, 16)] * d
                bh = tbuf[s, k, pl.ds(48, 16)] * d
                g1l = gbuf[s, k, pl.ds(32, 16)]
                g1h = gbuf[s, k, pl.ds(48, 16)]
                vxl = tbuf[s, k, pl.ds(64, 16)] * d
                vxh = tbuf[s, k, pl.ds(80, 16)] * d
                vyl = tbuf[s, k, pl.ds(96, 16)] * d
                vyh = tbuf[s, k, pl.ds(112, 16)] * d
                vzl = tbuf[s, k, pl.ds(128, 16)] * d
                vzh = tbuf[s, k, pl.ds(144, 16)] * d
                s0l = s0l + gbuf[s, k, pl.ds(0, 16)] * al
                s0h = s0h + gbuf[s, k, pl.ds(16, 16)] * ah
                sxl = sxl + g1l * vxl + gbuf[s, k, pl.ds(64, 16)] * bl
                sxh = sxh + g1h * vxh + gbuf[s, k, pl.ds(80, 16)] * bh
                syl = syl + g1l * vyl + gbuf[s, k, pl.ds(96, 16)] * bl
                syh = syh + g1h * vyh + gbuf[s, k, pl.ds(112, 16)] * bh
                szl = szl + g1l * vzl + gbuf[s, k, pl.ds(128, 16)] * bl
                szh = szh + g1h * vzh + gbuf[s, k, pl.ds(144, 16)] * bh
            return (s0l, s0h, sxl, sxh, syl, syh, szl, szh)
        return lax.fori_loop(0, 8, grp, carry)

    issue_idx(0, 0)
    wait_idx(0)
    issue_main(0, 0)
    issue_idx(1, 1)

    def half(g, s, n, carry):
        @pl.when(g < ng - 1)
        def _():
            wait_idx(n)
        wait_main(s)

        @pl.when(g < ng - 1)
        def _():
            issue_main(g + 1, n)

        @pl.when(g < ng - 2)
        def _():
            issue_idx(g + 2, s)
        return compute(s, carry)

    def body(i, carry):
        carry = half(2 * i, 0, 1, carry)
        return half(2 * i + 1, 1, 0, carry)

    z = jnp.zeros((16,), F32)
    accv = lax.fori_loop(0, ng // 2, body, (z,) * 8)
    for i, v in enumerate(accv):
        ob[pl.ds(i * 16, 16)] = v
    pltpu.sync_copy(ob, out.at[wid])


def _k_sum(npad, epad, esub, src2, dst2, gg, idt, tab):
    body = partial(_k_sum_body, npad, epad, esub)
    return pl.kernel(
        body,
        out_type=jax.ShapeDtypeStruct((NC * NSUB, 128), F32),
        mesh=_mesh(),
        compiler_params=pltpu.CompilerParams(use_tc_tiling_on_sc=False),
        scratch_types=[
            pltpu.VMEM((2, 1, 128), I32),
            pltpu.VMEM((2, 1, 128), I32),
            pltpu.VMEM((2, 128, 160), F32),
            pltpu.VMEM((2, 128, 160), F32),
            pltpu.VMEM((2, 128, 16), F32),
            pltpu.VMEM((128,), F32),
            pltpu.SemaphoreType.DMA,
            pltpu.SemaphoreType.DMA,
            pltpu.SemaphoreType.DMA,
            pltpu.SemaphoreType.DMA,
        ],
    )(src2, dst2, gg, idt, tab)


def _t_rad_body(n_edges, wr1, br1, wr2a, br2a, wr2c, br2c,
                ps, pd, ea, ga0, gb0, ga1, gb1, gc):
    i = pl.program_id(0)
    be = ps.shape[0]
    p_s = ps[...]
    p_d = pd[...]
    r = p_d[:, 0:3] - p_s[:, 0:3]
    d2 = jnp.sum(r * r, axis=1, keepdims=True) + 1e-8
    dist = jnp.sqrt(d2)
    rhat = r / dist
    rx = rhat[:, 0:1]
    ry = rhat[:, 1:2]
    rz = rhat[:, 2:3]
    eidx = i * be + lax.broadcasted_iota(I32, (be, 1), 0)
    valid = (eidx < n_edges).astype(F32)
    ef = jnp.concatenate([dist, ea[...]], axis=1)
    wr1a, br1a = wr1[...], br1[...]
    wr2aa, br2aa = wr2a[...], br2a[...]
    for l, oa, ob_ in ((0, ga0, gb0), (1, ga1, gb1)):
        h = jnp.maximum(jnp.dot(ef, wr1a[l], preferred_element_type=F32)
                        + br1a[l][None, :], 0.0)
        g = jnp.dot(h, wr2aa[l], preferred_element_type=F32) + br2aa[l][None, :]
        g = g * valid
        g0c, g1c, g2c = g[:, 0:16], g[:, 16:32], g[:, 32:48]
        oa[...] = jnp.concatenate([g1c, g2c * rx, g2c * ry], axis=1)
        ob_[...] = jnp.concatenate([g0c, g1c, g2c * rz], axis=1)
    h = jnp.maximum(jnp.dot(ef, wr1a[2], preferred_element_type=F32)
                    + br1a[2][None, :], 0.0)
    g = jnp.dot(h, wr2c[...], preferred_element_type=F32) + br2c[...][None, :]
    g = g * valid
    g2c = g[:, 64:96]
    gc[...] = jnp.concatenate([g[:, 0:64], g2c * rx, g2c * ry, g2c * rz],
                              axis=1)


def _t_rad(n_edges, epad, conv, ps, pd, eap):
    be = 2048
    grid = epad // be
    wr1 = jnp.stack([c["Wr1"] for c in conv])
    br1 = jnp.stack([c["br1"] for c in conv])
    wr2a = jnp.stack([conv[0]["Wr2"], conv[1]["Wr2"]])
    br2a = jnp.stack([conv[0]["br2"], conv[1]["br2"]])
    wr2c = conv[2]["Wr2"]
    br2c = conv[2]["br2"]
    full = lambda a: pl.BlockSpec(a.shape, lambda i: (0,) * a.ndim)
    eb = lambda w: pl.BlockSpec((be, w), lambda i: (i, 0))
    return pl.pallas_call(
        partial(_t_rad_body, n_edges),
        grid=(grid,),
        in_specs=[full(wr1), full(br1), full(wr2a), full(br2a), full(wr2c),
                  full(br2c), eb(8), eb(8), eb(4)],
        out_specs=[eb(48), eb(48), eb(48), eb(48), eb(160)],
        out_shape=[
            jax.ShapeDtypeStruct((epad, 48), F32),
            jax.ShapeDtypeStruct((epad, 48), F32),
            jax.ShapeDtypeStruct((epad, 48), F32),
            jax.ShapeDtypeStruct((epad, 48), F32),
            jax.ShapeDtypeStruct((epad, 160), F32),
        ],
    )(wr1, br1, wr2a, br2a, wr2c, br2c, ps, pd, eap)


def _t_tab0_body(w0, mv, ta, tb):
    w = w0[...]
    m = mv[...][:, 0:1]
    a = m * w[0][None, :]
    b = m * w[2][None, :]
    vx = mv[...][:, 1:2] * w[1][None, :]
    vy = mv[...][:, 2:3] * w[1][None, :]
    vz = mv[...][:, 3:4] * w[1][None, :]
    ta[...] = jnp.concatenate([vx, vy, b], axis=1)
    tb[...] = jnp.concatenate([a, vz, b], axis=1)


def _t_tab0(npad, bn, conv0, mv):
    grid = npad // bn
    w0 = jnp.stack([conv0["Ws"][0], conv0["Wv"][0], conv0["Wsv"][0]])
    return pl.pallas_call(
        _t_tab0_body,
        grid=(grid,),
        in_specs=[pl.BlockSpec(w0.shape, lambda i: (0, 0)),
                  pl.BlockSpec((bn, 4), lambda i: (i, 0))],
        out_specs=[pl.BlockSpec((bn, 48), lambda i: (i, 0))] * 2,
        out_shape=[jax.ShapeDtypeStruct((npad, 48), F32)] * 2,
    )(w0, mv)


def _t_node0_body(u0, wn, bn_, w1, mv, out0, idt, ta, tb, h0o, h1o):
    o = out0[...]
    u = u0[...]
    w = w1[...]
    ivd = idt[...]
    amx, amy = o[0, :, 0:16] * ivd, o[0, :, 16:32] * ivd
    am0, amz = o[1, :, 0:16] * ivd, o[1, :, 16:32] * ivd
    m = mv[...][:, 0:1]
    h0 = jnp.maximum(am0 + m * u[0][None, :], 0.0)
    h1x = amx + mv[...][:, 1:2] * u[1][None, :]
    h1y = amy + mv[...][:, 2:3] * u[1][None, :]
    h1z = amz + mv[...][:, 3:4] * u[1][None, :]
    nrm = jnp.sqrt(h1x * h1x + h1y * h1y + h1z * h1z + 1e-12)
    sc = jax.nn.sigmoid(jnp.dot(nrm, wn[...], preferred_element_type=F32)
                        + bn_[...])
    h1x, h1y, h1z = h1x * sc, h1y * sc, h1z * sc
    a = jnp.dot(h0, w[0], preferred_element_type=F32)
    b = jnp.dot(h0, w[1], preferred_element_type=F32)
    vx = jnp.dot(h1x, w[2], preferred_element_type=F32)
    vy = jnp.dot(h1y, w[2], preferred_element_type=F32)
    vz = jnp.dot(h1z, w[2], preferred_element_type=F32)
    ta[...] = jnp.concatenate([vx, vy, b], axis=1)
    tb[...] = jnp.concatenate([a, vz, b], axis=1)
    h0o[...] = h0
    h1o[...] = jnp.concatenate([h1x, h1y, h1z], axis=1)


def _t_node0(npad, bn, conv0, norm0, conv1, mv, out0, idt):
    grid = npad // bn
    u0 = jnp.stack([conv0["Us"][0], conv0["Uv"][0]])
    w1 = jnp.stack([conv1["Ws"], conv1["Wsv"], conv1["Wv"]])
    bn1 = norm0["bn"][None, :]
    full = lambda a: pl.BlockSpec(a.shape, lambda i: (0,) * a.ndim)
    return pl.pallas_call(
        _t_node0_body,
        grid=(grid,),
        in_specs=[full(u0), full(norm0["Wn"]), full(bn1), full(w1),
                  pl.BlockSpec((bn, 4), lambda i: (i, 0)),
                  pl.BlockSpec((NC, bn, 32), lambda i: (0, i, 0)),
                  pl.BlockSpec((bn, 16), lambda i: (i, 0))],
        out_specs=[pl.BlockSpec((bn, 48), lambda i: (i, 0)),
                   pl.BlockSpec((bn, 48), lambda i: (i, 0)),
                   pl.BlockSpec((bn, 16), lambda i: (i, 0)),
                   pl.BlockSpec((bn, 48), lambda i: (i, 0))],
        out_shape=[jax.ShapeDtypeStruct((npad, 48), F32),
                   jax.ShapeDtypeStruct((npad, 48), F32),
                   jax.ShapeDtypeStruct((npad, 16), F32),
                   jax.ShapeDtypeStruct((npad, 48), F32)],
    )(u0, norm0["Wn"], bn1, w1, mv, out0, idt)


def _t_node1_body(u1, wn, bn_, w2, h0i, h1i, out1, idt, tab, sh0, sh1):
    i = pl.program_id(0)
    o = out1[...]
    u = u1[...]
    w = w2[...]
    ivd = idt[...]
    amx, amy = o[0, :, 0:16] * ivd, o[0, :, 16:32] * ivd
    am0, amz = o[1, :, 0:16] * ivd, o[1, :, 16:32] * ivd
    h1p = h1i[...]
    h0 = jnp.maximum(am0 + jnp.dot(h0i[...], u[0], preferred_element_type=F32),
                     0.0)
    h1x = amx + jnp.dot(h1p[:, 0:16], u[1], preferred_element_type=F32)
    h1y = amy + jnp.dot(h1p[:, 16:32], u[1], preferred_element_type=F32)
    h1z = amz + jnp.dot(h1p[:, 32:48], u[1], preferred_element_type=F32)
    nrm = jnp.sqrt(h1x * h1x + h1y * h1y + h1z * h1z + 1e-12)
    sc = jax.nn.sigmoid(jnp.dot(nrm, wn[...], preferred_element_type=F32)
                        + bn_[...])
    h1x, h1y, h1z = h1x * sc, h1y * sc, h1z * sc
    a = jnp.dot(h0, w[0], preferred_element_type=F32)
    b = jnp.dot(h0, w[1], preferred_element_type=F32)
    vx = jnp.dot(h1x, w[2], preferred_element_type=F32)
    vy = jnp.dot(h1y, w[2], preferred_element_type=F32)
    vz = jnp.dot(h1z, w[2], preferred_element_type=F32)
    tab[...] = jnp.concatenate([a, b, vx, vy, vz], axis=1)

    @pl.when(i == 0)
    def _():
        sh0[...] = jnp.zeros_like(sh0)
        sh1[...] = jnp.zeros_like(sh1)
    sh0[...] += jnp.sum(h0, axis=0, keepdims=True)
    sh1[...] += jnp.concatenate(
        [jnp.sum(h1x, axis=0, keepdims=True),
         jnp.sum(h1y, axis=0, keepdims=True),
         jnp.sum(h1z, axis=0, keepdims=True)], axis=1)


def _t_node1(npad, bn, conv1, norm1, conv2, h0, h1, out1, idt):
    grid = npad // bn
    u1 = jnp.stack([conv1["Us"], conv1["Uv"]])
    w2 = jnp.stack([conv2["Ws"], conv2["Wsv"], conv2["Wv"]])
    bn1 = norm1["bn"][None, :]
    full = lambda a: pl.BlockSpec(a.shape, lambda i: (0,) * a.ndim)
    return pl.pallas_call(
        _t_node1_body,
        grid=(grid,),
        in_specs=[full(u1), full(norm1["Wn"]), full(bn1), full(w2),
                  pl.BlockSpec((bn, 16), lambda i: (i, 0)),
                  pl.BlockSpec((bn, 48), lambda i: (i, 0)),
                  pl.BlockSpec((NC, bn, 32), lambda i: (0, i, 0)),
                  pl.BlockSpec((bn, 16), lambda i: (i, 0))],
        out_specs=[pl.BlockSpec((bn, 160), lambda i: (i, 0)),
                   pl.BlockSpec((1, 16), lambda i: (0, 0)),
                   pl.BlockSpec((1, 48), lambda i: (0, 0))],
        out_shape=[jax.ShapeDtypeStruct((npad, 160), F32),
                   jax.ShapeDtypeStruct((1, 16), F32),
                   jax.ShapeDtypeStruct((1, 48), F32)],
    )(u1, norm1["Wn"], bn1, w2, h0, h1, out1, idt)


def _t_fin_body(n_nodes, u2, f0, f1, o2, sh0, sh1, o0, o1):
    s = jnp.sum(o2[...], axis=0)
    u, fa, fb = u2[...], f0[...], f1[...]
    inv_n = 1.0 / n_nodes
    mh0 = (s[0:32][None, :]
           + jnp.dot(sh0[...], u[0], preferred_element_type=F32)) * inv_n
    o0[...] = jnp.dot(jnp.dot(mh0, fa[0], preferred_element_type=F32),
                      fb[0], preferred_element_type=F32)
    rows = []
    for i in range(3):
        mh1 = (s[32 + 32 * i:64 + 32 * i][None, :]
               + jnp.dot(sh1[...][:, 16 * i:16 * i + 16], u[1],
                         preferred_element_type=F32)) * inv_n
        rows.append(jnp.dot(jnp.dot(mh1, fa[1], preferred_element_type=F32),
                            fb[1], preferred_element_type=F32))
    o1[...] = jnp.concatenate(rows, axis=0)


def _t_fin(n_nodes, conv2, fc, o2, sh0, sh1):
    u2 = jnp.stack([conv2["Us"], conv2["Uv"]])
    f0 = jnp.stack([fc["Wf0_s"], fc["Wf0_v"]])
    f1 = jnp.stack([fc["Wf1_s"], fc["Wf1_v"]])
    full = lambda a: pl.BlockSpec(a.shape, lambda: (0,) * a.ndim)
    return pl.pallas_call(
        partial(_t_fin_body, float(n_nodes)),
        in_specs=[full(u2), full(f0), full(f1), full(o2), full(sh0), full(sh1)],
        out_specs=[full(jnp.zeros((1, 2))), full(jnp.zeros((3, 2)))],
        out_shape=[jax.ShapeDtypeStruct((1, 2), F32),
                   jax.ShapeDtypeStruct((3, 2), F32)],
    )(u2, f0, f1, o2, sh0, sh1)


def kernel(pos, mass, velocity, edge_attr, edge_index, params):
    n_nodes = pos.shape[0]
    n_edges = edge_index.shape[1]
    npad, nrows, epad, esub32 = _dims(n_nodes, n_edges)
    esub16 = esub32 * NC
    bn = nrows

    conv = params["conv"]
    norm = params["norm"]
    fc = params["fc"]

    src = edge_index[0].astype(I32)
    dst = edge_index[1].astype(I32)
    pad_e = epad - n_edges
    srcp = jnp.concatenate([src, jnp.full((pad_e,), n_nodes, I32)])
    dstp = jnp.concatenate([dst, jnp.full((pad_e,), n_nodes, I32)])
    src2 = srcp.reshape(epad // 128, 128)
    dst2 = dstp.reshape(epad // 128, 128)
    posp = jnp.zeros((npad, 8), F32).at[:n_nodes, 0:3].set(pos)
    mv = jnp.zeros((npad, 4), F32).at[:n_nodes, 0:1].set(mass)
    mv = mv.at[:n_nodes, 1:4].set(velocity)
    eap = jnp.zeros((epad, 4), F32).at[:n_edges].set(edge_attr)

    ps, pd, idt = _k_geo(n_nodes, npad, nrows, epad, esub16, posp, srcp, dstp)
    ga0, gb0, ga1, gb1, gc = _t_rad(n_edges, epad, conv, ps, pd, eap)
    ta0, tb0 = _t_tab0(npad, bn, conv[0], mv)
    out0 = _k_edge(npad, nrows, epad, esub16, src2, dst2, ga0, gb0, ta0, tb0)
    ta1, tb1, h0, h1 = _t_node0(npad, bn, conv[0], norm[0], conv[1], mv, out0,
                                idt)
    out1 = _k_edge(npad, nrows, epad, esub16, src2, dst2, ga1, gb1, ta1, tb1)
    tab, sh0, sh1 = _t_node1(npad, bn, conv[1], norm[1], conv[2], h0, h1, out1,
                             idt)
    o2 = _k_sum(npad, epad, esub32, src2, dst2, gc, idt, tab)
    o0, o1 = _t_fin(n_nodes, conv[2], fc, o2, sh0, sh1)
    return o0, jnp.transpose(o1)[None]
